# baseline (speedup 1.0000x reference)
import functools

import jax
import jax.numpy as jnp
from jax import lax
from jax.experimental import pallas as pl
from jax.experimental.pallas import tpu as pltpu
from jax.experimental.pallas import tpu_sc as plsc

B = 1024
L = 50
H = 64
V = 100000
TOPK = 64
BB = 256
GRID = B // BB


def _gru_argmax_body(x_ref, inpT_ref, wih_ref, whh_ref, bih_ref, bhh_ref,
                     lin_ref, linb_ref, out_ref, gi_ref, hall_ref):
    i = pl.program_id(0)
    x2 = x_ref[...].reshape(BB * L, H)
    gi = jnp.dot(x2, wih_ref[...], preferred_element_type=jnp.float32) + bih_ref[...]
    gi_ref[...] = gi.reshape(BB, L, 3 * H)

    def step(t, h):
        gi_t = gi_ref[:, t]
        gh = jnp.dot(h, whh_ref[...], preferred_element_type=jnp.float32) + bhh_ref[...]
        r = jax.nn.sigmoid(gi_t[:, 0:H] + gh[:, 0:H])
        z = jax.nn.sigmoid(gi_t[:, H:2 * H] + gh[:, H:2 * H])
        n = jnp.tanh(gi_t[:, 2 * H:3 * H] + r * gh[:, 2 * H:3 * H])
        h2 = (1.0 - z) * n + z * h
        hall_ref[t] = h2
        return h2

    lax.fori_loop(0, L, step, jnp.zeros((BB, H), jnp.float32))

    logits = jnp.dot(hall_ref[...].reshape(L * BB, H), lin_ref[...],
                     preferred_element_type=jnp.float32) + linb_ref[...]
    maxv = jnp.max(logits, axis=-1, keepdims=True)
    col = lax.broadcasted_iota(jnp.int32, logits.shape, 1)
    sel = jnp.min(jnp.where(logits == maxv, col, TOPK), axis=-1)
    inp_blk = inpT_ref[:, pl.ds(i * BB, BB)]
    out_ref[:, pl.ds(i * BB, BB)] = inp_blk * TOPK + sel.reshape(L, BB)


def _tc_sel_indices(x, inpT, wihT, whhT, bih, bhh, linT, linb):
    return pl.pallas_call(
        _gru_argmax_body,
        grid=(GRID,),
        in_specs=[
            pl.BlockSpec((BB, L, H), lambda i: (i, 0, 0)),
            pl.BlockSpec((L, B), lambda i: (0, 0)),
            pl.BlockSpec((H, 3 * H), lambda i: (0, 0)),
            pl.BlockSpec((H, 3 * H), lambda i: (0, 0)),
            pl.BlockSpec((1, 3 * H), lambda i: (0, 0)),
            pl.BlockSpec((1, 3 * H), lambda i: (0, 0)),
            pl.BlockSpec((H, TOPK), lambda i: (0, 0)),
            pl.BlockSpec((1, TOPK), lambda i: (0, 0)),
        ],
        out_specs=pl.BlockSpec((L, B), lambda i: (0, 0)),
        out_shape=jax.ShapeDtypeStruct((L, B), jnp.int32),
        scratch_shapes=[
            pltpu.VMEM((BB, L, 3 * H), jnp.float32),
            pltpu.VMEM((L, BB, H), jnp.float32),
        ],
        compiler_params=pltpu.CompilerParams(
            dimension_semantics=("arbitrary",),
        ),
    )(x, inpT, wihT, whhT, bih, bhh, linT, linb)


def _make_sc_gather(nc, ns):
    nw = nc * ns
    per_w = (B * L) // nw
    ch = 80
    nchunk = per_w // ch
    mesh = plsc.VectorSubcoreMesh(core_axis_name="c", subcore_axis_name="s")

    @functools.partial(
        pl.kernel,
        out_type=jax.ShapeDtypeStruct((B * L, H), jnp.float32),
        mesh=mesh,
        scratch_types=[
            pltpu.VMEM((per_w,), jnp.int32),
            pltpu.VMEM((ch,), jnp.int32),
            pltpu.VMEM((ch, H), jnp.float32),
            pltpu.SemaphoreType.DMA,
            pltpu.SemaphoreType.DMA,
        ],
        compiler_params=pltpu.CompilerParams(use_tc_tiling_on_sc=False),
    )
    def sc_kernel(fidx_hbm, sim_hbm, emb_hbm, out_hbm, idx_v, words_v, rows_v,
                  sem1, sem2):
        wid = lax.axis_index("s") * nc + lax.axis_index("c")
        base = pl.multiple_of(wid * per_w, 8)
        pltpu.sync_copy(fidx_hbm.at[pl.ds(base, per_w)], idx_v)

        def chunk(c, carry):
            off = pl.multiple_of(c * ch, 8)
            pltpu.async_copy(sim_hbm.at[idx_v.at[pl.ds(off, ch)]], words_v, sem1).wait()
            pltpu.async_copy(emb_hbm.at[words_v], rows_v, sem2).wait()
            pltpu.sync_copy(rows_v, out_hbm.at[pl.ds(base + off, ch)])
            return carry

        lax.fori_loop(0, nchunk, chunk, 0)

    return sc_kernel


def kernel(inp, hidden_out, similar_words, max_replacements_ratio, emb_table,
           dgru_Wih, dgru_Whh, dgru_bih, dgru_bhh,
           sgru_Wih, sgru_Whh, sgru_bih, sgru_bhh,
           conv_w, conv_b, lin_w, lin_b):
    inpT = jnp.swapaxes(inp.astype(jnp.int32), 0, 1)
    fidxT = _tc_sel_indices(
        hidden_out, inpT,
        sgru_Wih.T, sgru_Whh.T,
        sgru_bih.reshape(1, 3 * H), sgru_bhh.reshape(1, 3 * H),
        lin_w.T, lin_b.reshape(1, TOPK),
    )
    fidx = jnp.swapaxes(fidxT, 0, 1).reshape(B * L)
    info = plsc.get_sparse_core_info()
    sc = _make_sc_gather(info.num_cores, info.num_subcores)
    out = sc(fidx, similar_words.reshape(V * TOPK).astype(jnp.int32), emb_table)
    return out.reshape(B, L, H)


# speedup vs baseline: 1.5257x; 1.5257x over previous
import functools

import jax
import jax.numpy as jnp
from jax import lax
from jax.experimental import pallas as pl
from jax.experimental.pallas import tpu as pltpu
from jax.experimental.pallas import tpu_sc as plsc

B = 1024
L = 50
H = 64
V = 100000
TOPK = 64
BB = 512
GRID = B // BB


def _gru_argmax_body(xT_ref, inpT_ref, wih_ref, whh_ref, bih_ref, bhh_ref,
                     lin_ref, linb_ref, out_ref):
    wih = wih_ref[...]
    whh = whh_ref[...]
    bih = bih_ref[...]
    bhh = bhh_ref[...]
    lin = lin_ref[...]
    linb = linb_ref[...]

    def step(t, h):
        gi = jnp.dot(xT_ref[t], wih, preferred_element_type=jnp.float32) + bih
        gh = jnp.dot(h, whh, preferred_element_type=jnp.float32) + bhh
        r = jax.nn.sigmoid(gi[:, 0:H] + gh[:, 0:H])
        z = jax.nn.sigmoid(gi[:, H:2 * H] + gh[:, H:2 * H])
        n = jnp.tanh(gi[:, 2 * H:3 * H] + r * gh[:, 2 * H:3 * H])
        h2 = (1.0 - z) * n + z * h
        logits = jnp.dot(h2, lin, preferred_element_type=jnp.float32) + linb
        maxv = jnp.max(logits, axis=-1, keepdims=True)
        col = lax.broadcasted_iota(jnp.int32, logits.shape, 1)
        sel = jnp.min(jnp.where(logits == maxv, col, TOPK), axis=-1)
        out_ref[t] = inpT_ref[t] * TOPK + sel
        return h2

    lax.fori_loop(0, L, step, jnp.zeros((B, H), jnp.float32))


def _tc_sel_indices(xT, inpT, wihT, whhT, bih, bhh, linT, linb):
    return pl.pallas_call(
        _gru_argmax_body,
        out_shape=jax.ShapeDtypeStruct((L, B), jnp.int32),
    )(xT, inpT, wihT, whhT, bih, bhh, linT, linb)


def _make_sc_gather(nc, ns):
    nw = nc * ns
    per_w = (B * L) // nw
    ch = 80
    nchunk = per_w // ch
    mesh = plsc.VectorSubcoreMesh(core_axis_name="c", subcore_axis_name="s")

    @functools.partial(
        pl.kernel,
        out_type=jax.ShapeDtypeStruct((B * L, H), jnp.float32),
        mesh=mesh,
        scratch_types=[
            pltpu.VMEM((per_w,), jnp.int32),
            pltpu.VMEM((ch,), jnp.int32),
            pltpu.VMEM((ch,), jnp.int32),
            pltpu.VMEM((ch, TOPK), jnp.int32),
            pltpu.VMEM((ch, H), jnp.float32),
            pltpu.SemaphoreType.DMA,
            pltpu.SemaphoreType.DMA,
        ],
        compiler_params=pltpu.CompilerParams(use_tc_tiling_on_sc=False,
                                             needs_layout_passes=False),
    )
    def sc_kernel(fidx_hbm, sim_hbm, emb_hbm, out_hbm, idx_v, rowidx_v, words_v,
                  simrows_v, rows_v, sem1, sem2):
        wid = lax.axis_index("s") * nc + lax.axis_index("c")
        base = pl.multiple_of(wid * per_w, 8)
        pltpu.sync_copy(fidx_hbm.at[pl.ds(base, per_w)], idx_v)

        def chunk(c, carry):
            off = pl.multiple_of(c * ch, 8)
            for k in range(ch // 16):
                fi = idx_v[pl.ds(off + 16 * k, 16)]
                rowidx_v[pl.ds(16 * k, 16)] = lax.shift_right_logical(fi, 6)
            pltpu.async_copy(sim_hbm.at[rowidx_v], simrows_v, sem1).wait()
            for k in range(ch // 16):
                fi = idx_v[pl.ds(off + 16 * k, 16)]
                rloc = lax.broadcasted_iota(jnp.int32, (16,), 0) + (16 * k)
                words_v[pl.ds(16 * k, 16)] = plsc.load_gather(
                    simrows_v, [rloc, fi & (TOPK - 1)])
            pltpu.async_copy(emb_hbm.at[words_v], rows_v, sem2).wait()
            pltpu.sync_copy(rows_v, out_hbm.at[pl.ds(base + off, ch)])
            return carry

        lax.fori_loop(0, nchunk, chunk, 0)

    return sc_kernel


def kernel(inp, hidden_out, similar_words, max_replacements_ratio, emb_table,
           dgru_Wih, dgru_Whh, dgru_bih, dgru_bhh,
           sgru_Wih, sgru_Whh, sgru_bih, sgru_bhh,
           conv_w, conv_b, lin_w, lin_b):
    xT = jnp.swapaxes(hidden_out, 0, 1)
    inpT = jnp.swapaxes(inp.astype(jnp.int32), 0, 1)
    fidxT = _tc_sel_indices(
        xT, inpT,
        sgru_Wih.T, sgru_Whh.T,
        sgru_bih.reshape(1, 3 * H), sgru_bhh.reshape(1, 3 * H),
        lin_w.T, lin_b.reshape(1, TOPK),
    )
    fidx = jnp.swapaxes(fidxT, 0, 1).reshape(B * L)
    info = plsc.get_sparse_core_info()
    sc = _make_sc_gather(info.num_cores, info.num_subcores)
    out = sc(fidx, similar_words.astype(jnp.int32), emb_table)
    return out.reshape(B, L, H)


# speedup vs baseline: 1.8123x; 1.1879x over previous
import functools

import jax
import jax.numpy as jnp
from jax import lax
from jax.experimental import pallas as pl
from jax.experimental.pallas import tpu as pltpu
from jax.experimental.pallas import tpu_sc as plsc

B = 1024
L = 50
H = 64
V = 100000
TOPK = 64
BB = 512
GRID = B // BB


def _gru_argmax_body(xT_ref, inpT_ref, wih_ref, whh_ref, bih_ref, bhh_ref,
                     lin_ref, linb_ref, out_ref, hall_ref):
    wih = wih_ref[...]
    whh = whh_ref[...]
    bih = bih_ref[...]
    bhh = bhh_ref[...]
    lin = lin_ref[...]
    linb = linb_ref[...]

    def gru_step(t, h):
        gi = jnp.dot(xT_ref[t], wih, preferred_element_type=jnp.float32) + bih
        gh = jnp.dot(h, whh, preferred_element_type=jnp.float32) + bhh
        r = jax.nn.sigmoid(gi[:, 0:H] + gh[:, 0:H])
        z = jax.nn.sigmoid(gi[:, H:2 * H] + gh[:, H:2 * H])
        n = jnp.tanh(gi[:, 2 * H:3 * H] + r * gh[:, 2 * H:3 * H])
        h2 = (1.0 - z) * n + z * h
        hall_ref[t] = h2
        return h2

    def step2(i, h):
        h = gru_step(2 * i, h)
        return gru_step(2 * i + 1, h)

    lax.fori_loop(0, L // 2, step2, jnp.zeros((B, H), jnp.float32))

    TCH = 5
    for tc in range(0, L, TCH):
        hs = hall_ref[tc:tc + TCH].reshape(TCH * B, H)
        logits = jnp.dot(hs, lin, preferred_element_type=jnp.float32) + linb
        maxv = jnp.max(logits, axis=-1, keepdims=True)
        col = lax.broadcasted_iota(jnp.int32, logits.shape, 1)
        sel = jnp.min(jnp.where(logits == maxv, col, TOPK), axis=-1).reshape(TCH, B)
        out_ref[tc:tc + TCH] = inpT_ref[tc:tc + TCH] * TOPK + sel


def _tc_sel_indices(xT, inpT, wihT, whhT, bih, bhh, linT, linb):
    return pl.pallas_call(
        _gru_argmax_body,
        out_shape=jax.ShapeDtypeStruct((L, B), jnp.int32),
        scratch_shapes=[pltpu.VMEM((L, B, H), jnp.float32)],
    )(xT, inpT, wihT, whhT, bih, bhh, linT, linb)


def _make_sc_gather(nc, ns):
    nw = nc * ns
    per_w = (B * L) // nw
    ch = 80
    nchunk = per_w // ch
    mesh = plsc.VectorSubcoreMesh(core_axis_name="c", subcore_axis_name="s")

    @functools.partial(
        pl.kernel,
        out_type=jax.ShapeDtypeStruct((B * L, H), jnp.float32),
        mesh=mesh,
        scratch_types=[
            pltpu.VMEM((per_w,), jnp.int32),
            pltpu.VMEM((ch,), jnp.int32),
            pltpu.VMEM((ch,), jnp.int32),
            pltpu.VMEM((ch, TOPK), jnp.int32),
            pltpu.VMEM((ch, H), jnp.float32),
            pltpu.SemaphoreType.DMA,
            pltpu.SemaphoreType.DMA,
        ],
        compiler_params=pltpu.CompilerParams(use_tc_tiling_on_sc=False,
                                             needs_layout_passes=False),
    )
    def sc_kernel(fidx_hbm, sim_hbm, emb_hbm, out_hbm, idx_v, rowidx_v, words_v,
                  simrows_v, rows_v, sem1, sem2):
        wid = lax.axis_index("s") * nc + lax.axis_index("c")
        base = pl.multiple_of(wid * per_w, 8)
        pltpu.sync_copy(fidx_hbm.at[pl.ds(base, per_w)], idx_v)

        def chunk(c, carry):
            off = pl.multiple_of(c * ch, 8)
            for k in range(ch // 16):
                fi = idx_v[pl.ds(off + 16 * k, 16)]
                rowidx_v[pl.ds(16 * k, 16)] = lax.shift_right_logical(fi, 6)
            pltpu.async_copy(sim_hbm.at[rowidx_v], simrows_v, sem1).wait()
            for k in range(ch // 16):
                fi = idx_v[pl.ds(off + 16 * k, 16)]
                rloc = lax.broadcasted_iota(jnp.int32, (16,), 0) + (16 * k)
                words_v[pl.ds(16 * k, 16)] = plsc.load_gather(
                    simrows_v, [rloc, fi & (TOPK - 1)])
            pltpu.async_copy(emb_hbm.at[words_v], rows_v, sem2).wait()
            pltpu.sync_copy(rows_v, out_hbm.at[pl.ds(base + off, ch)])
            return carry

        lax.fori_loop(0, nchunk, chunk, 0)

    return sc_kernel


def kernel(inp, hidden_out, similar_words, max_replacements_ratio, emb_table,
           dgru_Wih, dgru_Whh, dgru_bih, dgru_bhh,
           sgru_Wih, sgru_Whh, sgru_bih, sgru_bhh,
           conv_w, conv_b, lin_w, lin_b):
    xT = jnp.swapaxes(hidden_out, 0, 1)
    inpT = jnp.swapaxes(inp.astype(jnp.int32), 0, 1)
    fidxT = _tc_sel_indices(
        xT, inpT,
        sgru_Wih.T, sgru_Whh.T,
        sgru_bih.reshape(1, 3 * H), sgru_bhh.reshape(1, 3 * H),
        lin_w.T, lin_b.reshape(1, TOPK),
    )
    fidx = jnp.swapaxes(fidxT, 0, 1).reshape(B * L)
    info = plsc.get_sparse_core_info()
    sc = _make_sc_gather(info.num_cores, info.num_subcores)
    out = sc(fidx, similar_words.astype(jnp.int32), emb_table)
    return out.reshape(B, L, H)


# speedup vs baseline: 2.0531x; 1.1329x over previous
import functools

import jax
import jax.numpy as jnp
from jax import lax
from jax.experimental import pallas as pl
from jax.experimental.pallas import tpu as pltpu
from jax.experimental.pallas import tpu_sc as plsc

B = 1024
L = 50
H = 64
V = 100000
TOPK = 64
BB = 512
GRID = B // BB


def _gru_argmax_body(xT_ref, inpT_ref, wih_ref, whh_ref, bih_ref, bhh_ref,
                     lin_ref, linb_ref, out_ref, hall_ref):
    wih = wih_ref[...]
    whh = whh_ref[...]
    bih = bih_ref[...]
    bhh = bhh_ref[...]
    lin = lin_ref[...]
    linb = linb_ref[...]

    def gru_step(t, h):
        gi = jnp.dot(xT_ref[t], wih, preferred_element_type=jnp.float32) + bih
        gh = jnp.dot(h, whh, preferred_element_type=jnp.float32) + bhh
        r = jax.nn.sigmoid(gi[:, 0:H] + gh[:, 0:H])
        z = jax.nn.sigmoid(gi[:, H:2 * H] + gh[:, H:2 * H])
        n = jnp.tanh(gi[:, 2 * H:3 * H] + r * gh[:, 2 * H:3 * H])
        h2 = (1.0 - z) * n + z * h
        hall_ref[t] = h2
        return h2

    def step2(i, h):
        h = gru_step(2 * i, h)
        return gru_step(2 * i + 1, h)

    lax.fori_loop(0, L // 2, step2, jnp.zeros((B, H), jnp.float32))

    TCH = 5
    for tc in range(0, L, TCH):
        hs = hall_ref[tc:tc + TCH].reshape(TCH * B, H)
        logits = jnp.dot(hs, lin, preferred_element_type=jnp.float32) + linb
        maxv = jnp.max(logits, axis=-1, keepdims=True)
        col = lax.broadcasted_iota(jnp.int32, logits.shape, 1)
        sel = jnp.min(jnp.where(logits == maxv, col, TOPK), axis=-1).reshape(TCH, B)
        out_ref[tc:tc + TCH] = inpT_ref[tc:tc + TCH] * TOPK + sel


def _tc_sel_indices(xT, inpT, wihT, whhT, bih, bhh, linT, linb):
    return pl.pallas_call(
        _gru_argmax_body,
        out_shape=jax.ShapeDtypeStruct((L, B), jnp.int32),
        scratch_shapes=[pltpu.VMEM((L, B, H), jnp.float32)],
    )(xT, inpT, wihT, whhT, bih, bhh, linT, linb)


def _make_sc_gather(nc, ns):
    nw = nc * ns
    per_w = (B * L) // nw
    ch = 80
    nchunk = per_w // ch
    mesh = plsc.VectorSubcoreMesh(core_axis_name="c", subcore_axis_name="s")

    @functools.partial(
        pl.kernel,
        out_type=jax.ShapeDtypeStruct((B * L, H), jnp.float32),
        mesh=mesh,
        scratch_types=[
            pltpu.VMEM((per_w,), jnp.int32),
            pltpu.VMEM((per_w,), jnp.int32),
            pltpu.VMEM((per_w, H), jnp.float32),
            pltpu.SemaphoreType.DMA,
            pltpu.SemaphoreType.DMA,
        ],
        compiler_params=pltpu.CompilerParams(use_tc_tiling_on_sc=False,
                                             needs_layout_passes=False),
    )
    def sc_kernel(fidx_hbm, sim_hbm, emb_hbm, out_hbm, idx_v, words_v, rows_v,
                  semw, seme):
        wid = lax.axis_index("s") * nc + lax.axis_index("c")
        base = pl.multiple_of(wid * per_w, 8)
        pltpu.sync_copy(fidx_hbm.at[pl.ds(base, per_w)], idx_v)

        def words_copy(c):
            off = pl.multiple_of(c * ch, 8)
            return pltpu.make_async_copy(
                sim_hbm.at[idx_v.at[pl.ds(off, ch)]],
                words_v.at[pl.ds(off, ch)], semw)

        def fire_w(c, carry):
            words_copy(c).start()
            return carry

        def drain_w(c, carry):
            words_copy(c).wait()
            return carry

        lax.fori_loop(0, nchunk, fire_w, 0)
        lax.fori_loop(0, nchunk, drain_w, 0)

        def rows_copy(c):
            off = pl.multiple_of(c * ch, 8)
            return pltpu.make_async_copy(
                emb_hbm.at[words_v.at[pl.ds(off, ch)]],
                rows_v.at[pl.ds(off, ch)], seme)

        def fire_e(c, carry):
            rows_copy(c).start()
            return carry

        def drain_e(c, carry):
            rows_copy(c).wait()
            return carry

        lax.fori_loop(0, nchunk, fire_e, 0)
        lax.fori_loop(0, nchunk, drain_e, 0)

        pltpu.sync_copy(rows_v, out_hbm.at[pl.ds(base, per_w)])

    return sc_kernel


def kernel(inp, hidden_out, similar_words, max_replacements_ratio, emb_table,
           dgru_Wih, dgru_Whh, dgru_bih, dgru_bhh,
           sgru_Wih, sgru_Whh, sgru_bih, sgru_bhh,
           conv_w, conv_b, lin_w, lin_b):
    xT = jnp.swapaxes(hidden_out, 0, 1)
    inpT = jnp.swapaxes(inp.astype(jnp.int32), 0, 1)
    fidxT = _tc_sel_indices(
        xT, inpT,
        sgru_Wih.T, sgru_Whh.T,
        sgru_bih.reshape(1, 3 * H), sgru_bhh.reshape(1, 3 * H),
        lin_w.T, lin_b.reshape(1, TOPK),
    )
    fidx = jnp.swapaxes(fidxT, 0, 1).reshape(B * L)
    info = plsc.get_sparse_core_info()
    sc = _make_sc_gather(info.num_cores, info.num_subcores)
    out = sc(fidx, similar_words.reshape(V * TOPK).astype(jnp.int32), emb_table)
    return out.reshape(B, L, H)


# speedup vs baseline: 2.1155x; 1.0304x over previous
import functools

import jax
import jax.numpy as jnp
from jax import lax
from jax.experimental import pallas as pl
from jax.experimental.pallas import tpu as pltpu
from jax.experimental.pallas import tpu_sc as plsc

B = 1024
L = 50
H = 64
V = 100000
TOPK = 64
BB = 512
GRID = B // BB


def _gru_argmax_body(xT_ref, inpT_ref, wih_ref, whh_ref, bih_ref, bhh_ref,
                     lin_ref, linb_ref, out_ref, hall_ref):
    wih = wih_ref[...]
    whh = whh_ref[...]
    bih = bih_ref[...]
    bhh = bhh_ref[...]
    lin = lin_ref[...]
    linb = linb_ref[...]

    def gru_step(t, h):
        gi = jnp.dot(xT_ref[t], wih, preferred_element_type=jnp.float32) + bih
        gh = jnp.dot(h, whh, preferred_element_type=jnp.float32) + bhh
        r = jax.nn.sigmoid(gi[:, 0:H] + gh[:, 0:H])
        z = jax.nn.sigmoid(gi[:, H:2 * H] + gh[:, H:2 * H])
        n = jnp.tanh(gi[:, 2 * H:3 * H] + r * gh[:, 2 * H:3 * H])
        h2 = (1.0 - z) * n + z * h
        hall_ref[t] = h2
        return h2

    def step5(i, h):
        for j in range(5):
            h = gru_step(5 * i + j, h)
        return h

    lax.fori_loop(0, L // 5, step5, jnp.zeros((B, H), jnp.float32))

    TCH = 5
    for tc in range(0, L, TCH):
        hs = hall_ref[tc:tc + TCH].reshape(TCH * B, H)
        logits = jnp.dot(hs, lin, preferred_element_type=jnp.float32) + linb
        maxv = jnp.max(logits, axis=-1, keepdims=True)
        col = lax.broadcasted_iota(jnp.int32, logits.shape, 1)
        sel = jnp.min(jnp.where(logits == maxv, col, TOPK), axis=-1).reshape(TCH, B)
        out_ref[tc:tc + TCH] = inpT_ref[tc:tc + TCH] * TOPK + sel


def _tc_sel_indices(xT, inpT, wihT, whhT, bih, bhh, linT, linb):
    return pl.pallas_call(
        _gru_argmax_body,
        out_shape=jax.ShapeDtypeStruct((L, B), jnp.int32),
        scratch_shapes=[pltpu.VMEM((L, B, H), jnp.float32)],
    )(xT, inpT, wihT, whhT, bih, bhh, linT, linb)


def _make_sc_gather(nc, ns):
    nw = nc * ns
    bw = B // nw
    mesh = plsc.VectorSubcoreMesh(core_axis_name="c", subcore_axis_name="s")

    @functools.partial(
        pl.kernel,
        out_type=jax.ShapeDtypeStruct((B, L, H), jnp.float32),
        mesh=mesh,
        scratch_types=[
            pltpu.VMEM((bw, L), jnp.int32),
            pltpu.VMEM((bw, L), jnp.int32),
            pltpu.VMEM((bw, L, H), jnp.float32),
            pltpu.SemaphoreType.DMA,
            pltpu.SemaphoreType.DMA,
        ],
        compiler_params=pltpu.CompilerParams(use_tc_tiling_on_sc=False,
                                             needs_layout_passes=False),
    )
    def sc_kernel(fidx_hbm, sim_hbm, emb_hbm, out_hbm, idx_v, words_v, rows_v,
                  semw, seme):
        wid = lax.axis_index("s") * nc + lax.axis_index("c")
        base = pl.multiple_of(wid * bw, 8)
        pltpu.sync_copy(fidx_hbm.at[pl.ds(base, bw)], idx_v)

        def words_copy(c):
            return pltpu.make_async_copy(
                sim_hbm.at[idx_v.at[c]], words_v.at[c], semw)

        def fire_w(c, carry):
            words_copy(c).start()
            return carry

        def drain_w(c, carry):
            words_copy(c).wait()
            return carry

        lax.fori_loop(0, bw, fire_w, 0)
        lax.fori_loop(0, bw, drain_w, 0)

        def rows_copy(c):
            return pltpu.make_async_copy(
                emb_hbm.at[words_v.at[c]], rows_v.at[c], seme)

        def fire_e(c, carry):
            rows_copy(c).start()
            return carry

        def drain_e(c, carry):
            rows_copy(c).wait()
            return carry

        lax.fori_loop(0, bw, fire_e, 0)
        lax.fori_loop(0, bw, drain_e, 0)

        pltpu.sync_copy(rows_v, out_hbm.at[pl.ds(base, bw)])

    return sc_kernel


def kernel(inp, hidden_out, similar_words, max_replacements_ratio, emb_table,
           dgru_Wih, dgru_Whh, dgru_bih, dgru_bhh,
           sgru_Wih, sgru_Whh, sgru_bih, sgru_bhh,
           conv_w, conv_b, lin_w, lin_b):
    xT = jnp.swapaxes(hidden_out, 0, 1)
    inpT = jnp.swapaxes(inp.astype(jnp.int32), 0, 1)
    fidxT = _tc_sel_indices(
        xT, inpT,
        sgru_Wih.T, sgru_Whh.T,
        sgru_bih.reshape(1, 3 * H), sgru_bhh.reshape(1, 3 * H),
        lin_w.T, lin_b.reshape(1, TOPK),
    )
    fidx = jnp.swapaxes(fidxT, 0, 1)
    info = plsc.get_sparse_core_info()
    sc = _make_sc_gather(info.num_cores, info.num_subcores)
    return sc(fidx, similar_words.reshape(V * TOPK).astype(jnp.int32), emb_table)


# speedup vs baseline: 2.1800x; 1.0305x over previous
import functools

import jax
import jax.numpy as jnp
from jax import lax
from jax.experimental import pallas as pl
from jax.experimental.pallas import tpu as pltpu
from jax.experimental.pallas import tpu_sc as plsc

B = 1024
L = 50
H = 64
V = 100000
TOPK = 64
BB = 512
GRID = B // BB


def _gru_argmax_body(xT_ref, inpT_ref, wih_ref, whh_ref, bih_ref, bhh_ref,
                     lin_ref, linb_ref, out_ref, hall_ref):
    wih = wih_ref[...]
    whh = whh_ref[...]
    bih = bih_ref[...]
    bhh = bhh_ref[...]
    lin = lin_ref[...]
    linb = linb_ref[...]

    def gru_step(t, h):
        gi = jnp.dot(xT_ref[t], wih, preferred_element_type=jnp.float32) + bih
        gh = jnp.dot(h, whh, preferred_element_type=jnp.float32) + bhh
        rz = jax.nn.sigmoid(gi[:, 0:2 * H] + gh[:, 0:2 * H])
        r = rz[:, 0:H]
        z = rz[:, H:2 * H]
        n = jnp.tanh(gi[:, 2 * H:3 * H] + r * gh[:, 2 * H:3 * H])
        h2 = (1.0 - z) * n + z * h
        hall_ref[t] = h2
        return h2

    def step5(i, h):
        for j in range(5):
            h = gru_step(5 * i + j, h)
        return h

    lax.fori_loop(0, L // 5, step5, jnp.zeros((B, H), jnp.float32))

    TCH = 5
    for tc in range(0, L, TCH):
        hs = hall_ref[tc:tc + TCH].reshape(TCH * B, H)
        logits = jnp.dot(hs, lin, preferred_element_type=jnp.float32) + linb
        maxv = jnp.max(logits, axis=-1, keepdims=True)
        col = lax.broadcasted_iota(jnp.int32, logits.shape, 1).astype(jnp.float32)
        sel_f = jnp.min(jnp.where(logits == maxv, col, float(TOPK)), axis=-1)
        sel = sel_f.astype(jnp.int32).reshape(TCH, B)
        out_ref[tc:tc + TCH] = inpT_ref[tc:tc + TCH] * TOPK + sel


def _tc_sel_indices(xT, inpT, wihT, whhT, bih, bhh, linT, linb):
    return pl.pallas_call(
        _gru_argmax_body,
        out_shape=jax.ShapeDtypeStruct((L, B), jnp.int32),
        scratch_shapes=[pltpu.VMEM((L, B, H), jnp.float32)],
    )(xT, inpT, wihT, whhT, bih, bhh, linT, linb)


def _make_sc_gather(nc, ns):
    nw = nc * ns
    bw = B // nw
    mesh = plsc.VectorSubcoreMesh(core_axis_name="c", subcore_axis_name="s")

    @functools.partial(
        pl.kernel,
        out_type=jax.ShapeDtypeStruct((B, L, H), jnp.float32),
        mesh=mesh,
        scratch_types=[
            pltpu.VMEM((bw, L), jnp.int32),
            pltpu.VMEM((bw, L), jnp.int32),
            pltpu.VMEM((bw, L, H), jnp.float32),
            pltpu.SemaphoreType.DMA,
            pltpu.SemaphoreType.DMA,
        ],
        compiler_params=pltpu.CompilerParams(use_tc_tiling_on_sc=False,
                                             needs_layout_passes=False),
    )
    def sc_kernel(fidx_hbm, sim_hbm, emb_hbm, out_hbm, idx_v, words_v, rows_v,
                  semw, seme):
        wid = lax.axis_index("s") * nc + lax.axis_index("c")
        base = pl.multiple_of(wid * bw, 8)
        pltpu.sync_copy(fidx_hbm.at[pl.ds(base, bw)], idx_v)

        def words_copy(c):
            return pltpu.make_async_copy(
                sim_hbm.at[idx_v.at[c]], words_v.at[c], semw)

        def fire_w(c, carry):
            words_copy(c).start()
            return carry

        def drain_w(c, carry):
            words_copy(c).wait()
            return carry

        lax.fori_loop(0, bw, fire_w, 0)
        lax.fori_loop(0, bw, drain_w, 0)

        def rows_copy(c):
            return pltpu.make_async_copy(
                emb_hbm.at[words_v.at[c]], rows_v.at[c], seme)

        def fire_e(c, carry):
            rows_copy(c).start()
            return carry

        def drain_e(c, carry):
            rows_copy(c).wait()
            return carry

        lax.fori_loop(0, bw, fire_e, 0)
        lax.fori_loop(0, bw, drain_e, 0)

        pltpu.sync_copy(rows_v, out_hbm.at[pl.ds(base, bw)])

    return sc_kernel


def kernel(inp, hidden_out, similar_words, max_replacements_ratio, emb_table,
           dgru_Wih, dgru_Whh, dgru_bih, dgru_bhh,
           sgru_Wih, sgru_Whh, sgru_bih, sgru_bhh,
           conv_w, conv_b, lin_w, lin_b):
    xT = jnp.swapaxes(hidden_out, 0, 1)
    inpT = jnp.swapaxes(inp.astype(jnp.int32), 0, 1)
    fidxT = _tc_sel_indices(
        xT, inpT,
        sgru_Wih.T, sgru_Whh.T,
        sgru_bih.reshape(1, 3 * H), sgru_bhh.reshape(1, 3 * H),
        lin_w.T, lin_b.reshape(1, TOPK),
    )
    fidx = jnp.swapaxes(fidxT, 0, 1)
    info = plsc.get_sparse_core_info()
    sc = _make_sc_gather(info.num_cores, info.num_subcores)
    return sc(fidx, similar_words.reshape(V * TOPK).astype(jnp.int32), emb_table)


# speedup vs baseline: 2.1844x; 1.0020x over previous
import functools

import jax
import jax.numpy as jnp
from jax import lax
from jax.experimental import pallas as pl
from jax.experimental.pallas import tpu as pltpu
from jax.experimental.pallas import tpu_sc as plsc

B = 1024
L = 50
H = 64
V = 100000
TOPK = 64
BB = 512
GRID = B // BB


def _gru_argmax_body(xT_ref, inpT_ref, wih_ref, whh_ref, bih_ref, bhh_ref,
                     lin_ref, linb_ref, out_ref, hall_ref):
    wih = wih_ref[...]
    whh = whh_ref[...]
    bih = bih_ref[...]
    bhh = bhh_ref[...]
    lin = lin_ref[...]
    linb = linb_ref[...]

    def gru_step(t, h):
        gi = jnp.dot(xT_ref[t], wih, preferred_element_type=jnp.float32) + bih
        gh = jnp.dot(h, whh, preferred_element_type=jnp.float32) + bhh
        rz = jax.nn.sigmoid(gi[:, 0:2 * H] + gh[:, 0:2 * H])
        r = rz[:, 0:H]
        z = rz[:, H:2 * H]
        n = jnp.tanh(gi[:, 2 * H:3 * H] + r * gh[:, 2 * H:3 * H])
        h2 = (1.0 - z) * n + z * h
        hall_ref[t] = h2
        return h2

    def step10(i, h):
        for j in range(10):
            h = gru_step(10 * i + j, h)
        return h

    lax.fori_loop(0, L // 10, step10, jnp.zeros((B, H), jnp.float32))

    TCH = 5
    for tc in range(0, L, TCH):
        hs = hall_ref[tc:tc + TCH].reshape(TCH * B, H)
        logits = jnp.dot(hs, lin, preferred_element_type=jnp.float32) + linb
        maxv = jnp.max(logits, axis=-1, keepdims=True)
        col = lax.broadcasted_iota(jnp.int32, logits.shape, 1).astype(jnp.float32)
        sel_f = jnp.min(jnp.where(logits == maxv, col, float(TOPK)), axis=-1)
        sel = sel_f.astype(jnp.int32).reshape(TCH, B)
        out_ref[tc:tc + TCH] = inpT_ref[tc:tc + TCH] * TOPK + sel


def _tc_sel_indices(xT, inpT, wihT, whhT, bih, bhh, linT, linb):
    return pl.pallas_call(
        _gru_argmax_body,
        out_shape=jax.ShapeDtypeStruct((L, B), jnp.int32),
        scratch_shapes=[pltpu.VMEM((L, B, H), jnp.float32)],
    )(xT, inpT, wihT, whhT, bih, bhh, linT, linb)


def _make_sc_gather(nc, ns):
    nw = nc * ns
    bw = B // nw
    mesh = plsc.VectorSubcoreMesh(core_axis_name="c", subcore_axis_name="s")

    @functools.partial(
        pl.kernel,
        out_type=jax.ShapeDtypeStruct((B, L, H), jnp.float32),
        mesh=mesh,
        scratch_types=[
            pltpu.VMEM((bw, L), jnp.int32),
            pltpu.VMEM((bw, L), jnp.int32),
            pltpu.VMEM((bw, L, H), jnp.float32),
            pltpu.SemaphoreType.DMA,
            pltpu.SemaphoreType.DMA,
        ],
        compiler_params=pltpu.CompilerParams(use_tc_tiling_on_sc=False,
                                             needs_layout_passes=False),
    )
    def sc_kernel(fidx_hbm, sim_hbm, emb_hbm, out_hbm, idx_v, words_v, rows_v,
                  semw, seme):
        wid = lax.axis_index("s") * nc + lax.axis_index("c")
        base = pl.multiple_of(wid * bw, 8)
        pltpu.sync_copy(fidx_hbm.at[pl.ds(base, bw)], idx_v)

        def words_copy(c):
            return pltpu.make_async_copy(
                sim_hbm.at[idx_v.at[c]], words_v.at[c], semw)

        def fire_w(c, carry):
            words_copy(c).start()
            return carry

        def drain_w(c, carry):
            words_copy(c).wait()
            return carry

        lax.fori_loop(0, bw, fire_w, 0)
        lax.fori_loop(0, bw, drain_w, 0)

        def rows_copy(c):
            return pltpu.make_async_copy(
                emb_hbm.at[words_v.at[c]], rows_v.at[c], seme)

        def fire_e(c, carry):
            rows_copy(c).start()
            return carry

        def drain_e(c, carry):
            rows_copy(c).wait()
            return carry

        lax.fori_loop(0, bw, fire_e, 0)
        lax.fori_loop(0, bw, drain_e, 0)

        pltpu.sync_copy(rows_v, out_hbm.at[pl.ds(base, bw)])

    return sc_kernel


def kernel(inp, hidden_out, similar_words, max_replacements_ratio, emb_table,
           dgru_Wih, dgru_Whh, dgru_bih, dgru_bhh,
           sgru_Wih, sgru_Whh, sgru_bih, sgru_bhh,
           conv_w, conv_b, lin_w, lin_b):
    xT = jnp.swapaxes(hidden_out, 0, 1)
    inpT = jnp.swapaxes(inp.astype(jnp.int32), 0, 1)
    fidxT = _tc_sel_indices(
        xT, inpT,
        sgru_Wih.T, sgru_Whh.T,
        sgru_bih.reshape(1, 3 * H), sgru_bhh.reshape(1, 3 * H),
        lin_w.T, lin_b.reshape(1, TOPK),
    )
    fidx = jnp.swapaxes(fidxT, 0, 1)
    info = plsc.get_sparse_core_info()
    sc = _make_sc_gather(info.num_cores, info.num_subcores)
    return sc(fidx, similar_words.reshape(V * TOPK).astype(jnp.int32), emb_table)
